# Initial kernel scaffold; baseline (speedup 1.0000x reference)
import jax
import jax.numpy as jnp
from jax.experimental import pallas as pl


def kernel(x, hx_list, W, b):
    raise NotImplementedError("write your pallas kernel here")


# speedup vs baseline: 1.2989x; 1.2989x over previous
import jax
import jax.numpy as jnp
from jax.experimental import pallas as pl

_MEM_SLOTS = 262144


def _bbpm_kernel(x_ref, w_ref, b_ref, out_ref):
    B, T, D = x_ref.shape
    x = x_ref[...]
    s = jnp.sum(x * 1000.0, axis=-1)
    h = jnp.floor(s).astype(jnp.int32)
    slots = jnp.mod(h, _MEM_SLOTS)
    slots_f = slots.astype(jnp.float32)
    qslots_f = slots_f[:, T - 1]

    qb = jnp.broadcast_to(qslots_f[:, None, None], (B, B, T))
    sb = jnp.broadcast_to(slots_f[None, :, :], (B, B, T))
    t_idx = jax.lax.broadcasted_iota(jnp.int32, (B, B, T), 2)
    mask = jnp.where((qb == sb) & (t_idx < (T - 1)), 1.0, 0.0)
    mask = mask.reshape(B, B * T)

    writes = x.reshape(B * T, D)
    retrieved = jax.lax.dot_general(
        mask, writes, (((1,), (0,)), ((), ())),
        precision=jax.lax.Precision.HIGHEST,
        preferred_element_type=jnp.float32)

    out = jax.lax.dot_general(
        retrieved, w_ref[...], (((1,), (1,)), ((), ())),
        precision=jax.lax.Precision.HIGHEST,
        preferred_element_type=jnp.float32)
    out_ref[...] = out + b_ref[...][None, :]


def kernel(x, hx_list, W, b):
    del hx_list
    B, T, D = x.shape
    return pl.pallas_call(
        _bbpm_kernel,
        out_shape=jax.ShapeDtypeStruct((B, D), x.dtype),
    )(x, W, b)
